# Initial kernel scaffold; baseline (speedup 1.0000x reference)
import jax
import jax.numpy as jnp
from jax.experimental import pallas as pl


def kernel(x, edge_index, batch, emb, W1, b1, W2, b2, Wfc, bfc):
    raise NotImplementedError("write your pallas kernel here")


# speedup vs baseline: 16.4628x; 16.4628x over previous
import functools

import jax
import jax.numpy as jnp
from jax import lax
from jax.experimental import pallas as pl
from jax.experimental.pallas import tpu as pltpu
from jax.experimental.pallas import tpu_sc as plsc

N = 50000
E = 800000
VOCAB = 100000
EMBED = 64
H1 = 128
H2 = 64
G = 1024

NC = 2
NS = 16
LANE = 128

NP = 53248
NODE_CH = 13
EP = 802816
ECH = 6272
ECH_PT32 = ECH // 32
ECH_PT16 = ECH // 16
NDUMP = N
GP = 1152
GDUMP = G
ROWS_PT = NP // NS
ZR = 416
GRP = 8

BLK = 2048
NBLK = NP // BLK

_mesh = plsc.VectorSubcoreMesh(core_axis_name="c", subcore_axis_name="s")


def _zero_fill_1d(buf, nwords):
    z = jnp.zeros((16,), jnp.float32)

    def body(i, _):
        buf[pl.ds(i * 16, 16)] = z
        return 0

    lax.fori_loop(0, nwords // 16, body, 0)


def _zero_fill_2d(buf, nrows):
    z = jnp.zeros((16,), jnp.float32)

    def body(i, _):
        buf[i, pl.ds(0, 16)] = z
        return 0

    lax.fori_loop(0, nrows, body, 0)


def _make_prep():
    @functools.partial(
        pl.kernel,
        out_type=(
            jax.ShapeDtypeStruct((NP, EMBED), jnp.float32),
            jax.ShapeDtypeStruct((NC, NP), jnp.float32),
            jax.ShapeDtypeStruct((NC, GP), jnp.float32),
        ),
        mesh=_mesh,
        compiler_params=pltpu.CompilerParams(use_tc_tiling_on_sc=False),
        scratch_types=(
            pltpu.VMEM((LANE,), jnp.int32),
            pltpu.VMEM((LANE, EMBED), jnp.float32),
            pltpu.VMEM((LANE,), jnp.float32),
            pltpu.VMEM((ROWS_PT,), jnp.float32),
            pltpu.VMEM_SHARED((NP,), jnp.float32),
            pltpu.VMEM_SHARED((GP,), jnp.float32),
            pltpu.SemaphoreType.DMA,
        ),
    )
    def prep(emb, xi, dstp, batchp, h0_out, deg_out, cnt_out,
             idxb, rows, onesb, zb, dega, cnta, sem):
        c = lax.axis_index("c")
        s = lax.axis_index("s")
        wid = c * NS + s

        _zero_fill_1d(zb, ROWS_PT)
        o = jnp.ones((16,), jnp.float32)
        for i in range(8):
            onesb[pl.ds(i * 16, 16)] = o

        pltpu.sync_copy(zb, dega.at[pl.ds(s * ROWS_PT, ROWS_PT)])

        @pl.when(s == 0)
        def _():
            pltpu.sync_copy(zb.at[pl.ds(0, GP)], cnta)

        plsc.subcore_barrier()

        def deg_body(j, _):
            row = wid * ECH_PT32 + j
            pltpu.sync_copy(dstp.at[row], idxb)
            pltpu.sync_copy(onesb, dega.at[idxb], add=True)
            return 0

        lax.fori_loop(0, ECH_PT32, deg_body, 0)

        def cnt_body(j, _):
            row = wid * NODE_CH + j
            pltpu.sync_copy(batchp.at[row], idxb)
            pltpu.sync_copy(onesb, cnta.at[idxb], add=True)
            return 0

        lax.fori_loop(0, NODE_CH, cnt_body, 0)

        def emb_body(j, _):
            row = wid * NODE_CH + j
            pltpu.sync_copy(xi.at[row], idxb)
            pltpu.async_copy(emb.at[idxb], rows, sem).wait()
            pltpu.sync_copy(rows, h0_out.at[pl.ds(row * LANE, LANE)])
            return 0

        lax.fori_loop(0, NODE_CH, emb_body, 0)

        plsc.subcore_barrier()

        pltpu.sync_copy(dega.at[pl.ds(s * ROWS_PT, ROWS_PT)],
                        deg_out.at[c, pl.ds(s * ROWS_PT, ROWS_PT)])

        @pl.when(s == 0)
        def _():
            pltpu.sync_copy(cnta, cnt_out.at[c])

    return prep


def _make_agg():
    @functools.partial(
        pl.kernel,
        out_type=jax.ShapeDtypeStruct((NC, NP, 32), jnp.float32),
        mesh=_mesh,
        compiler_params=pltpu.CompilerParams(use_tc_tiling_on_sc=False),
        scratch_types=(
            pltpu.VMEM((GRP, LANE), jnp.int32),
            pltpu.VMEM((GRP, LANE), jnp.int32),
            pltpu.VMEM((GRP, LANE, 16), jnp.float32),
            pltpu.VMEM((ZR, 16), jnp.float32),
            pltpu.VMEM_SHARED((NP, 16), jnp.float32),
            pltpu.SemaphoreType.DMA,
            pltpu.SemaphoreType.DMA,
        ),
    )
    def agg(tlo2, thi2, srcp, dstp, u_out,
            srcb, dstb, rows, zb, acc, gsem, ssem):
        c = lax.axis_index("c")
        s = lax.axis_index("s")

        _zero_fill_2d(zb, ZR)

        def run_edges(table, q):
            def grp_body(g, _):
                base = s * ECH_PT16 + g * GRP
                pltpu.sync_copy(srcp.at[pl.ds(base, GRP)], srcb)
                pltpu.sync_copy(dstp.at[pl.ds(base, GRP)], dstb)
                for k in range(GRP):
                    for l in range(LANE // 16):
                        v = srcb[k, pl.ds(l * 16, 16)]
                        srcb[k, pl.ds(l * 16, 16)] = v * 2 + q
                gds = []
                for k in range(GRP):
                    gds.append(pltpu.async_copy(
                        table.at[srcb.at[k]], rows.at[k], gsem))
                for d in gds:
                    d.wait()
                sds = []
                for k in range(GRP):
                    sds.append(pltpu.async_copy(
                        rows.at[k], acc.at[dstb.at[k]], ssem, add=True))
                for d in sds:
                    d.wait()
                return 0

            lax.fori_loop(0, ECH_PT16 // GRP, grp_body, 0)

        for q in range(2):
            for k in range(8):
                pltpu.sync_copy(zb, acc.at[pl.ds(s * ROWS_PT + k * ZR, ZR)])
            plsc.subcore_barrier()

            @pl.when(c == 0)
            def _():
                run_edges(tlo2, q)

            @pl.when(c == 1)
            def _():
                run_edges(thi2, q)

            plsc.subcore_barrier()

            pltpu.sync_copy(
                acc.at[pl.ds(s * ROWS_PT, ROWS_PT)],
                u_out.at[c, pl.ds(s * ROWS_PT, ROWS_PT), pl.ds(16 * q, 16)])
            plsc.subcore_barrier()

    return agg


def _tc_scale_body(h0_ref, d0_ref, d1_ref, flo_ref, fhi_ref, dinv_ref):
    d = d0_ref[...] + d1_ref[...] + 1.0
    dv = lax.rsqrt(d)
    fs = h0_ref[...] * dv[:, None]
    flo_ref[...] = fs[:, :32]
    fhi_ref[...] = fs[:, 32:]
    dinv_ref[...] = dv


def _tc_scale(h0, deg0, deg1):
    return pl.pallas_call(
        _tc_scale_body,
        grid=(NBLK,),
        in_specs=[
            pl.BlockSpec((BLK, EMBED), lambda i: (i, 0)),
            pl.BlockSpec((BLK,), lambda i: (i,)),
            pl.BlockSpec((BLK,), lambda i: (i,)),
        ],
        out_specs=[
            pl.BlockSpec((BLK, 32), lambda i: (i, 0)),
            pl.BlockSpec((BLK, 32), lambda i: (i, 0)),
            pl.BlockSpec((BLK,), lambda i: (i,)),
        ],
        out_shape=[
            jax.ShapeDtypeStruct((NP, 32), jnp.float32),
            jax.ShapeDtypeStruct((NP, 32), jnp.float32),
            jax.ShapeDtypeStruct((NP,), jnp.float32),
        ],
    )(h0, deg0, deg1)


def _tc_mid_body(ulo_ref, uhi_ref, flo_ref, fhi_ref, dinv_ref,
                 w1_ref, b1_ref, w2_ref, tlo_ref, thi_ref):
    u = jnp.concatenate([ulo_ref[...], uhi_ref[...]], axis=1)
    fs = jnp.concatenate([flo_ref[...], fhi_ref[...]], axis=1)
    dv = dinv_ref[...]
    g1 = (u + fs) * dv[:, None]
    h1 = jnp.dot(g1, w1_ref[...], preferred_element_type=jnp.float32)
    h1 = jnp.maximum(h1 + b1_ref[...][None, :], 0.0)
    t = jnp.dot(h1, w2_ref[...], preferred_element_type=jnp.float32)
    ts = t * dv[:, None]
    tlo_ref[...] = ts[:, :32]
    thi_ref[...] = ts[:, 32:]


def _tc_mid(ulo, uhi, flo, fhi, dinv, W1, b1, W2):
    return pl.pallas_call(
        _tc_mid_body,
        grid=(NBLK,),
        in_specs=[
            pl.BlockSpec((BLK, 32), lambda i: (i, 0)),
            pl.BlockSpec((BLK, 32), lambda i: (i, 0)),
            pl.BlockSpec((BLK, 32), lambda i: (i, 0)),
            pl.BlockSpec((BLK, 32), lambda i: (i, 0)),
            pl.BlockSpec((BLK,), lambda i: (i,)),
            pl.BlockSpec((EMBED, H1), lambda i: (0, 0)),
            pl.BlockSpec((H1,), lambda i: (0,)),
            pl.BlockSpec((H1, H2), lambda i: (0, 0)),
        ],
        out_specs=[
            pl.BlockSpec((BLK, 32), lambda i: (i, 0)),
            pl.BlockSpec((BLK, 32), lambda i: (i, 0)),
        ],
        out_shape=[
            jax.ShapeDtypeStruct((NP, 32), jnp.float32),
            jax.ShapeDtypeStruct((NP, 32), jnp.float32),
        ],
    )(ulo, uhi, flo, fhi, dinv, W1, b1, W2)


def _tc_pool_body(ulo_ref, uhi_ref, tlo_ref, thi_ref, dinv_ref, b2_ref,
                  batch_ref, c0_ref, c1_ref, wfc_ref, bfc_ref,
                  out_ref, segacc):
    i = pl.program_id(0)

    @pl.when(i == 0)
    def _():
        segacc[...] = jnp.zeros_like(segacc)

    u = jnp.concatenate([ulo_ref[...], uhi_ref[...]], axis=1)
    ts = jnp.concatenate([tlo_ref[...], thi_ref[...]], axis=1)
    g2 = (u + ts) * dinv_ref[...][:, None]
    h2 = jnp.maximum(g2 + b2_ref[...][None, :], 0.0)
    b = batch_ref[0, 0, :]
    seg_ids = lax.broadcasted_iota(jnp.int32, (G, BLK), 0)
    mask = (b[None, :] == seg_ids).astype(jnp.float32)
    segacc[...] += jnp.dot(mask, h2, preferred_element_type=jnp.float32)

    @pl.when(i == pl.num_programs(0) - 1)
    def _():
        cnt = (c0_ref[...] + c1_ref[...])[:G]
        pooled = segacc[...] / jnp.maximum(cnt, 1.0)[:, None]
        res = jnp.dot(pooled, wfc_ref[...], preferred_element_type=jnp.float32)
        out_ref[...] = jax.nn.sigmoid(res + bfc_ref[0, 0])


def _tc_pool(ulo, uhi, tlo, thi, dinv, b2, batch3, c0, c1, Wfcp, bfc2):
    return pl.pallas_call(
        _tc_pool_body,
        grid=(NBLK,),
        in_specs=[
            pl.BlockSpec((BLK, 32), lambda i: (i, 0)),
            pl.BlockSpec((BLK, 32), lambda i: (i, 0)),
            pl.BlockSpec((BLK, 32), lambda i: (i, 0)),
            pl.BlockSpec((BLK, 32), lambda i: (i, 0)),
            pl.BlockSpec((BLK,), lambda i: (i,)),
            pl.BlockSpec((H2,), lambda i: (0,)),
            pl.BlockSpec((1, 1, BLK), lambda i: (i, 0, 0)),
            pl.BlockSpec((GP,), lambda i: (0,)),
            pl.BlockSpec((GP,), lambda i: (0,)),
            pl.BlockSpec((H2, LANE), lambda i: (0, 0)),
            pl.BlockSpec(memory_space=pltpu.SMEM),
        ],
        out_specs=pl.BlockSpec((G, LANE), lambda i: (0, 0)),
        out_shape=jax.ShapeDtypeStruct((G, LANE), jnp.float32),
        scratch_shapes=[pltpu.VMEM((G, H2), jnp.float32)],
    )(ulo, uhi, tlo, thi, dinv, b2, batch3, c0, c1, Wfcp, bfc2)


_prep = _make_prep()
_agg = _make_agg()


@jax.jit
def kernel(x, edge_index, batch, emb, W1, b1, W2, b2, Wfc, bfc):
    xi = x[:, 0].astype(jnp.int32)
    xi_p = jnp.concatenate(
        [xi, jnp.zeros((NP - N,), jnp.int32)]).reshape(NODE_CH * 32, LANE)
    batch_p = jnp.concatenate(
        [batch.astype(jnp.int32), jnp.full((NP - N,), GDUMP, jnp.int32)])
    batch2d = batch_p.reshape(NODE_CH * 32, LANE)
    batch3 = batch_p.reshape(NBLK, 1, BLK)
    src_p = jnp.concatenate(
        [edge_index[0].astype(jnp.int32), jnp.zeros((EP - E,), jnp.int32)]
    ).reshape(ECH, LANE)
    dst_p = jnp.concatenate(
        [edge_index[1].astype(jnp.int32), jnp.full((EP - E,), NDUMP, jnp.int32)]
    ).reshape(ECH, LANE)

    h0, deg, cnt = _prep(emb, xi_p, dst_p, batch2d)
    flo, fhi, dinv = _tc_scale(h0, deg[0], deg[1])
    u1 = _agg(flo.reshape(2 * NP, 16), fhi.reshape(2 * NP, 16), src_p, dst_p)
    tlo, thi = _tc_mid(u1[0], u1[1], flo, fhi, dinv, W1, b1, W2)
    u2 = _agg(tlo.reshape(2 * NP, 16), thi.reshape(2 * NP, 16), src_p, dst_p)
    Wfcp = jnp.pad(Wfc, ((0, 0), (0, LANE - 1)))
    bfc2 = bfc.reshape(1, 1)
    outp = _tc_pool(u2[0], u2[1], tlo, thi, dinv, b2, batch3,
                    cnt[0], cnt[1], Wfcp, bfc2)
    return outp[:, 0]


# speedup vs baseline: 22.1845x; 1.3475x over previous
import functools

import jax
import jax.numpy as jnp
from jax import lax
from jax.experimental import pallas as pl
from jax.experimental.pallas import tpu as pltpu
from jax.experimental.pallas import tpu_sc as plsc

N = 50000
E = 800000
VOCAB = 100000
EMBED = 64
H1 = 128
H2 = 64
G = 1024

NC = 2
NS = 16
LANE = 128

NP = 53248
NODE_CH = 13
EP = 802816
ECH = 6272
ECH_PT32 = ECH // 32
ECH_PT16 = ECH // 16
NDUMP = N
GP = 1152
GDUMP = G
ROWS_PT = NP // NS
ZR = 416
GRP = 8

BLK = 2048
NBLK = NP // BLK

_mesh = plsc.VectorSubcoreMesh(core_axis_name="c", subcore_axis_name="s")


def _zero_fill_1d(buf, nwords):
    z = jnp.zeros((16,), jnp.float32)

    def body(i, _):
        buf[pl.ds(i * 16, 16)] = z
        return 0

    lax.fori_loop(0, nwords // 16, body, 0)


def _zero_fill_2d(buf, nrows):
    z = jnp.zeros((16,), jnp.float32)

    def body(i, _):
        buf[i, pl.ds(0, 16)] = z
        return 0

    lax.fori_loop(0, nrows, body, 0)


def _make_prep():
    @functools.partial(
        pl.kernel,
        out_type=(
            jax.ShapeDtypeStruct((NP, EMBED), jnp.float32),
            jax.ShapeDtypeStruct((NC, NP), jnp.float32),
            jax.ShapeDtypeStruct((NC, GP), jnp.float32),
        ),
        mesh=_mesh,
        compiler_params=pltpu.CompilerParams(use_tc_tiling_on_sc=False),
        scratch_types=(
            pltpu.VMEM((8, LANE), jnp.int32),
            pltpu.VMEM((NODE_CH, LANE), jnp.int32),
            pltpu.VMEM((NODE_CH, LANE, EMBED), jnp.float32),
            pltpu.VMEM((LANE,), jnp.float32),
            pltpu.VMEM((ROWS_PT,), jnp.float32),
            pltpu.VMEM_SHARED((NP,), jnp.float32),
            pltpu.VMEM_SHARED((GP,), jnp.float32),
            pltpu.SemaphoreType.DMA,
            pltpu.SemaphoreType.DMA,
            pltpu.SemaphoreType.DMA,
        ),
    )
    def prep(emb, xi, dstp, batchp, h0_out, deg_out, cnt_out,
             idxb, idxb2, rows, onesb, zb, dega, cnta, gsem, ssem, wsem):
        c = lax.axis_index("c")
        s = lax.axis_index("s")
        wid = c * NS + s

        _zero_fill_1d(zb, ROWS_PT)
        o = jnp.ones((16,), jnp.float32)
        for i in range(8):
            onesb[pl.ds(i * 16, 16)] = o

        pltpu.sync_copy(zb, dega.at[pl.ds(s * ROWS_PT, ROWS_PT)])

        @pl.when(s == 0)
        def _():
            pltpu.sync_copy(zb.at[pl.ds(0, GP)], cnta)

        plsc.subcore_barrier()

        def deg_grp(base, n):
            pltpu.sync_copy(dstp.at[pl.ds(base, n)], idxb.at[pl.ds(0, n)])
            ds_ = []
            for k in range(n):
                ds_.append(pltpu.async_copy(
                    onesb, dega.at[idxb.at[k]], ssem, add=True))
            for d in ds_:
                d.wait()

        def deg_body(g, _):
            deg_grp(wid * ECH_PT32 + g * 8, 8)
            return 0

        lax.fori_loop(0, ECH_PT32 // 8, deg_body, 0)
        deg_grp(wid * ECH_PT32 + (ECH_PT32 // 8) * 8, ECH_PT32 % 8)

        pltpu.sync_copy(batchp.at[pl.ds(wid * NODE_CH, 8)], idxb)
        cds = []
        for k in range(8):
            cds.append(pltpu.async_copy(
                onesb, cnta.at[idxb.at[k]], ssem, add=True))
        for d in cds:
            d.wait()
        pltpu.sync_copy(batchp.at[pl.ds(wid * NODE_CH + 8, NODE_CH - 8)],
                        idxb.at[pl.ds(0, NODE_CH - 8)])
        cds = []
        for k in range(NODE_CH - 8):
            cds.append(pltpu.async_copy(
                onesb, cnta.at[idxb.at[k]], ssem, add=True))
        for d in cds:
            d.wait()

        pltpu.sync_copy(xi.at[pl.ds(wid * NODE_CH, NODE_CH)], idxb2)
        gds = []
        for k in range(NODE_CH):
            gds.append(pltpu.async_copy(
                emb.at[idxb2.at[k]], rows.at[k], gsem))
        for d in gds:
            d.wait()
        wds = []
        for k in range(NODE_CH):
            wds.append(pltpu.async_copy(
                rows.at[k],
                h0_out.at[pl.ds((wid * NODE_CH + k) * LANE, LANE)], wsem))
        for d in wds:
            d.wait()

        plsc.subcore_barrier()

        pltpu.sync_copy(dega.at[pl.ds(s * ROWS_PT, ROWS_PT)],
                        deg_out.at[c, pl.ds(s * ROWS_PT, ROWS_PT)])

        @pl.when(s == 0)
        def _():
            pltpu.sync_copy(cnta, cnt_out.at[c])

    return prep


def _make_agg():
    @functools.partial(
        pl.kernel,
        out_type=jax.ShapeDtypeStruct((NC, NP, 32), jnp.float32),
        mesh=_mesh,
        compiler_params=pltpu.CompilerParams(use_tc_tiling_on_sc=False),
        scratch_types=(
            pltpu.VMEM((2, GRP, LANE), jnp.int32),
            pltpu.VMEM((2, GRP, LANE), jnp.int32),
            pltpu.VMEM((2, GRP, LANE, 16), jnp.float32),
            pltpu.VMEM((ZR, 16), jnp.float32),
            pltpu.VMEM_SHARED((NP, 16), jnp.float32),
            pltpu.SemaphoreType.DMA,
            pltpu.SemaphoreType.DMA,
            pltpu.SemaphoreType.DMA,
        ),
    )
    def agg(tlo2, thi2, srcp, dstp, u_out,
            srcb, dstb, rows, zb, acc, gsem0, gsem1, ssem):
        c = lax.axis_index("c")
        s = lax.axis_index("s")
        NG = ECH_PT16 // GRP

        _zero_fill_2d(zb, ZR)

        def run_edges(table, q):
            def load_fire(g, slot):
                gsem = gsem0 if slot == 0 else gsem1
                base = s * ECH_PT16 + g * GRP
                pltpu.sync_copy(srcp.at[pl.ds(base, GRP)], srcb.at[slot])
                pltpu.sync_copy(dstp.at[pl.ds(base, GRP)], dstb.at[slot])
                for k in range(GRP):
                    for l in range(LANE // 16):
                        v = srcb[slot, k, pl.ds(l * 16, 16)]
                        srcb[slot, k, pl.ds(l * 16, 16)] = v * 2 + q
                for k in range(GRP):
                    pltpu.async_copy(
                        table.at[srcb.at[slot].at[k]], rows.at[slot].at[k],
                        gsem)

            def finish(g, slot, fire_next):
                gsem = gsem0 if slot == 0 else gsem1
                if fire_next:
                    @pl.when(g < NG - 1)
                    def _():
                        load_fire(g + 1, slot ^ 1)
                for k in range(GRP):
                    pltpu.make_async_copy(
                        table.at[srcb.at[slot].at[k]], rows.at[slot].at[k],
                        gsem).wait()
                sds = []
                for k in range(GRP):
                    sds.append(pltpu.async_copy(
                        rows.at[slot].at[k], acc.at[dstb.at[slot].at[k]],
                        ssem, add=True))
                for d in sds:
                    d.wait()

            load_fire(0, 0)

            def body(i, _):
                finish(2 * i, 0, True)
                finish(2 * i + 1, 1, True)
                return 0

            lax.fori_loop(0, (NG - 1) // 2, body, 0)
            finish(NG - 1, 0, False)

        for q in range(2):
            for k in range(8):
                pltpu.sync_copy(zb, acc.at[pl.ds(s * ROWS_PT + k * ZR, ZR)])
            plsc.subcore_barrier()

            @pl.when(c == 0)
            def _():
                run_edges(tlo2, q)

            @pl.when(c == 1)
            def _():
                run_edges(thi2, q)

            plsc.subcore_barrier()

            pltpu.sync_copy(
                acc.at[pl.ds(s * ROWS_PT, ROWS_PT)],
                u_out.at[c, pl.ds(s * ROWS_PT, ROWS_PT), pl.ds(16 * q, 16)])
            plsc.subcore_barrier()

    return agg


def _tc_scale_body(h0_ref, d0_ref, d1_ref, flo_ref, fhi_ref, dinv_ref):
    d = d0_ref[...] + d1_ref[...] + 1.0
    dv = lax.rsqrt(d)
    fs = h0_ref[...] * dv[:, None]
    flo_ref[...] = fs[:, :32]
    fhi_ref[...] = fs[:, 32:]
    dinv_ref[...] = dv


def _tc_scale(h0, deg0, deg1):
    return pl.pallas_call(
        _tc_scale_body,
        grid=(NBLK,),
        in_specs=[
            pl.BlockSpec((BLK, EMBED), lambda i: (i, 0)),
            pl.BlockSpec((BLK,), lambda i: (i,)),
            pl.BlockSpec((BLK,), lambda i: (i,)),
        ],
        out_specs=[
            pl.BlockSpec((BLK, 32), lambda i: (i, 0)),
            pl.BlockSpec((BLK, 32), lambda i: (i, 0)),
            pl.BlockSpec((BLK,), lambda i: (i,)),
        ],
        out_shape=[
            jax.ShapeDtypeStruct((NP, 32), jnp.float32),
            jax.ShapeDtypeStruct((NP, 32), jnp.float32),
            jax.ShapeDtypeStruct((NP,), jnp.float32),
        ],
    )(h0, deg0, deg1)


def _tc_mid_body(ulo_ref, uhi_ref, flo_ref, fhi_ref, dinv_ref,
                 w1_ref, b1_ref, w2_ref, tlo_ref, thi_ref):
    u = jnp.concatenate([ulo_ref[...], uhi_ref[...]], axis=1)
    fs = jnp.concatenate([flo_ref[...], fhi_ref[...]], axis=1)
    dv = dinv_ref[...]
    g1 = (u + fs) * dv[:, None]
    h1 = jnp.dot(g1, w1_ref[...], preferred_element_type=jnp.float32)
    h1 = jnp.maximum(h1 + b1_ref[...][None, :], 0.0)
    t = jnp.dot(h1, w2_ref[...], preferred_element_type=jnp.float32)
    ts = t * dv[:, None]
    tlo_ref[...] = ts[:, :32]
    thi_ref[...] = ts[:, 32:]


def _tc_mid(ulo, uhi, flo, fhi, dinv, W1, b1, W2):
    return pl.pallas_call(
        _tc_mid_body,
        grid=(NBLK,),
        in_specs=[
            pl.BlockSpec((BLK, 32), lambda i: (i, 0)),
            pl.BlockSpec((BLK, 32), lambda i: (i, 0)),
            pl.BlockSpec((BLK, 32), lambda i: (i, 0)),
            pl.BlockSpec((BLK, 32), lambda i: (i, 0)),
            pl.BlockSpec((BLK,), lambda i: (i,)),
            pl.BlockSpec((EMBED, H1), lambda i: (0, 0)),
            pl.BlockSpec((H1,), lambda i: (0,)),
            pl.BlockSpec((H1, H2), lambda i: (0, 0)),
        ],
        out_specs=[
            pl.BlockSpec((BLK, 32), lambda i: (i, 0)),
            pl.BlockSpec((BLK, 32), lambda i: (i, 0)),
        ],
        out_shape=[
            jax.ShapeDtypeStruct((NP, 32), jnp.float32),
            jax.ShapeDtypeStruct((NP, 32), jnp.float32),
        ],
    )(ulo, uhi, flo, fhi, dinv, W1, b1, W2)


def _tc_pool_body(ulo_ref, uhi_ref, tlo_ref, thi_ref, dinv_ref, b2_ref,
                  batch_ref, c0_ref, c1_ref, wfc_ref, bfc_ref,
                  out_ref, segacc):
    i = pl.program_id(0)

    @pl.when(i == 0)
    def _():
        segacc[...] = jnp.zeros_like(segacc)

    u = jnp.concatenate([ulo_ref[...], uhi_ref[...]], axis=1)
    ts = jnp.concatenate([tlo_ref[...], thi_ref[...]], axis=1)
    g2 = (u + ts) * dinv_ref[...][:, None]
    h2 = jnp.maximum(g2 + b2_ref[...][None, :], 0.0)
    b = batch_ref[0, 0, :]
    seg_ids = lax.broadcasted_iota(jnp.int32, (G, BLK), 0)
    mask = (b[None, :] == seg_ids).astype(jnp.float32)
    segacc[...] += jnp.dot(mask, h2, preferred_element_type=jnp.float32)

    @pl.when(i == pl.num_programs(0) - 1)
    def _():
        cnt = (c0_ref[...] + c1_ref[...])[:G]
        pooled = segacc[...] / jnp.maximum(cnt, 1.0)[:, None]
        res = jnp.dot(pooled, wfc_ref[...], preferred_element_type=jnp.float32)
        out_ref[...] = jax.nn.sigmoid(res + bfc_ref[0, 0])


def _tc_pool(ulo, uhi, tlo, thi, dinv, b2, batch3, c0, c1, Wfcp, bfc2):
    return pl.pallas_call(
        _tc_pool_body,
        grid=(NBLK,),
        in_specs=[
            pl.BlockSpec((BLK, 32), lambda i: (i, 0)),
            pl.BlockSpec((BLK, 32), lambda i: (i, 0)),
            pl.BlockSpec((BLK, 32), lambda i: (i, 0)),
            pl.BlockSpec((BLK, 32), lambda i: (i, 0)),
            pl.BlockSpec((BLK,), lambda i: (i,)),
            pl.BlockSpec((H2,), lambda i: (0,)),
            pl.BlockSpec((1, 1, BLK), lambda i: (i, 0, 0)),
            pl.BlockSpec((GP,), lambda i: (0,)),
            pl.BlockSpec((GP,), lambda i: (0,)),
            pl.BlockSpec((H2, LANE), lambda i: (0, 0)),
            pl.BlockSpec(memory_space=pltpu.SMEM),
        ],
        out_specs=pl.BlockSpec((G, LANE), lambda i: (0, 0)),
        out_shape=jax.ShapeDtypeStruct((G, LANE), jnp.float32),
        scratch_shapes=[pltpu.VMEM((G, H2), jnp.float32)],
    )(ulo, uhi, tlo, thi, dinv, b2, batch3, c0, c1, Wfcp, bfc2)


_prep = _make_prep()
_agg = _make_agg()


@jax.jit
def kernel(x, edge_index, batch, emb, W1, b1, W2, b2, Wfc, bfc):
    xi = x[:, 0].astype(jnp.int32)
    xi_p = jnp.concatenate(
        [xi, jnp.zeros((NP - N,), jnp.int32)]).reshape(NODE_CH * 32, LANE)
    batch_p = jnp.concatenate(
        [batch.astype(jnp.int32), jnp.full((NP - N,), GDUMP, jnp.int32)])
    batch2d = batch_p.reshape(NODE_CH * 32, LANE)
    batch3 = batch_p.reshape(NBLK, 1, BLK)
    src_p = jnp.concatenate(
        [edge_index[0].astype(jnp.int32), jnp.zeros((EP - E,), jnp.int32)]
    ).reshape(ECH, LANE)
    dst_p = jnp.concatenate(
        [edge_index[1].astype(jnp.int32), jnp.full((EP - E,), NDUMP, jnp.int32)]
    ).reshape(ECH, LANE)

    h0, deg, cnt = _prep(emb, xi_p, dst_p, batch2d)
    flo, fhi, dinv = _tc_scale(h0, deg[0], deg[1])
    u1 = _agg(flo.reshape(2 * NP, 16), fhi.reshape(2 * NP, 16), src_p, dst_p)
    tlo, thi = _tc_mid(u1[0], u1[1], flo, fhi, dinv, W1, b1, W2)
    u2 = _agg(tlo.reshape(2 * NP, 16), thi.reshape(2 * NP, 16), src_p, dst_p)
    Wfcp = jnp.pad(Wfc, ((0, 0), (0, LANE - 1)))
    bfc2 = bfc.reshape(1, 1)
    outp = _tc_pool(u2[0], u2[1], tlo, thi, dinv, b2, batch3,
                    cnt[0], cnt[1], Wfcp, bfc2)
    return outp[:, 0]


# speedup vs baseline: 22.9177x; 1.0331x over previous
import functools

import jax
import jax.numpy as jnp
from jax import lax
from jax.experimental import pallas as pl
from jax.experimental.pallas import tpu as pltpu
from jax.experimental.pallas import tpu_sc as plsc

N = 50000
E = 800000
VOCAB = 100000
EMBED = 64
H1 = 128
H2 = 64
G = 1024

NC = 2
NS = 16
LANE = 128

NP = 53248
NODE_CH = 13
EP = 802816
ECH = 6272
ECH_PT32 = ECH // 32
ECH_PT16 = ECH // 16
NDUMP = N
GP = 1152
GDUMP = G
ROWS_PT = NP // NS
ZR = 416
GRP = 8

BLK = 2048
NBLK = NP // BLK

_mesh = plsc.VectorSubcoreMesh(core_axis_name="c", subcore_axis_name="s")


def _zero_fill_1d(buf, nwords):
    z = jnp.zeros((16,), jnp.float32)

    def body(i, _):
        buf[pl.ds(i * 16, 16)] = z
        return 0

    lax.fori_loop(0, nwords // 16, body, 0)


def _zero_fill_2d(buf, nrows):
    z = jnp.zeros((16,), jnp.float32)

    def body(i, _):
        buf[i, pl.ds(0, 16)] = z
        return 0

    lax.fori_loop(0, nrows, body, 0)


def _make_prep():
    @functools.partial(
        pl.kernel,
        out_type=(
            jax.ShapeDtypeStruct((NP, EMBED), jnp.float32),
            jax.ShapeDtypeStruct((NC, NP), jnp.float32),
            jax.ShapeDtypeStruct((NC, GP), jnp.float32),
        ),
        mesh=_mesh,
        compiler_params=pltpu.CompilerParams(use_tc_tiling_on_sc=False),
        scratch_types=(
            pltpu.VMEM((8, LANE), jnp.int32),
            pltpu.VMEM((NODE_CH, LANE), jnp.int32),
            pltpu.VMEM((NODE_CH * LANE, EMBED), jnp.float32),
            pltpu.VMEM((LANE,), jnp.float32),
            pltpu.VMEM((ROWS_PT,), jnp.float32),
            pltpu.VMEM_SHARED((NP,), jnp.float32),
            pltpu.VMEM_SHARED((GP,), jnp.float32),
            pltpu.SemaphoreType.DMA,
            pltpu.SemaphoreType.DMA,
            pltpu.SemaphoreType.DMA,
        ),
    )
    def prep(emb, xi, dstp, batchp, h0_out, deg_out, cnt_out,
             idxb, idxb2, rows, onesb, zb, dega, cnta,
             gsem, ssem, wsem):
        c = lax.axis_index("c")
        s = lax.axis_index("s")
        wid = c * NS + s

        _zero_fill_1d(zb, ROWS_PT)
        o = jnp.ones((16,), jnp.float32)
        for i in range(8):
            onesb[pl.ds(i * 16, 16)] = o

        pltpu.sync_copy(zb, dega.at[pl.ds(s * ROWS_PT, ROWS_PT)])

        @pl.when(s == 0)
        def _():
            pltpu.sync_copy(zb.at[pl.ds(0, GP)], cnta)

        plsc.subcore_barrier()

        pltpu.sync_copy(xi.at[pl.ds(wid * NODE_CH, NODE_CH)], idxb2)
        gds = []
        for k in range(NODE_CH):
            gds.append(pltpu.async_copy(
                emb.at[idxb2.at[k]], rows.at[pl.ds(k * LANE, LANE)], gsem))

        def deg_grp(base, n):
            pltpu.sync_copy(dstp.at[pl.ds(base, n)], idxb.at[pl.ds(0, n)])
            ds_ = []
            for k in range(n):
                ds_.append(pltpu.async_copy(
                    onesb, dega.at[idxb.at[k]], ssem, add=True))
            for d in ds_:
                d.wait()

        def deg_body(g, _):
            deg_grp(wid * ECH_PT32 + g * 8, 8)
            return 0

        lax.fori_loop(0, ECH_PT32 // 8, deg_body, 0)
        deg_grp(wid * ECH_PT32 + (ECH_PT32 // 8) * 8, ECH_PT32 % 8)

        pltpu.sync_copy(batchp.at[pl.ds(wid * NODE_CH, 8)], idxb)
        cds = []
        for k in range(8):
            cds.append(pltpu.async_copy(
                onesb, cnta.at[idxb.at[k]], ssem, add=True))
        for d in cds:
            d.wait()
        pltpu.sync_copy(batchp.at[pl.ds(wid * NODE_CH + 8, NODE_CH - 8)],
                        idxb.at[pl.ds(0, NODE_CH - 8)])
        cds = []
        for k in range(NODE_CH - 8):
            cds.append(pltpu.async_copy(
                onesb, cnta.at[idxb.at[k]], ssem, add=True))
        for d in cds:
            d.wait()

        for d in gds:
            d.wait()

        plsc.subcore_barrier()

        pltpu.sync_copy(dega.at[pl.ds(s * ROWS_PT, ROWS_PT)],
                        deg_out.at[c, pl.ds(s * ROWS_PT, ROWS_PT)])

        wds = []
        for k in range(NODE_CH):
            wds.append(pltpu.async_copy(
                rows.at[pl.ds(k * LANE, LANE)],
                h0_out.at[pl.ds((wid * NODE_CH + k) * LANE, LANE)], wsem))
        for d in wds:
            d.wait()

        @pl.when(s == 0)
        def _():
            pltpu.sync_copy(cnta, cnt_out.at[c])

    return prep


def _make_agg():
    @functools.partial(
        pl.kernel,
        out_type=jax.ShapeDtypeStruct((NC, NP, 32), jnp.float32),
        mesh=_mesh,
        compiler_params=pltpu.CompilerParams(use_tc_tiling_on_sc=False),
        scratch_types=(
            pltpu.VMEM((2, GRP, LANE), jnp.int32),
            pltpu.VMEM((2, GRP, LANE), jnp.int32),
            pltpu.VMEM((2, GRP, LANE, 16), jnp.float32),
            pltpu.VMEM((ZR, 16), jnp.float32),
            pltpu.VMEM_SHARED((NP, 16), jnp.float32),
            pltpu.SemaphoreType.DMA,
            pltpu.SemaphoreType.DMA,
            pltpu.SemaphoreType.DMA,
        ),
    )
    def agg(tbl4, srcp4, dstp, u_out,
            srcb, dstb, rows, zb, acc, gsem0, gsem1, ssem):
        c = lax.axis_index("c")
        s = lax.axis_index("s")
        NG = ECH_PT16 // GRP

        _zero_fill_2d(zb, ZR)

        def run_edges(qoff):
            table = tbl4.at[pl.ds(qoff, 4 * NP - 3)]

            def load_fire(g, slot):
                gsem = gsem0 if slot == 0 else gsem1
                base = s * ECH_PT16 + g * GRP
                pltpu.sync_copy(srcp4.at[pl.ds(base, GRP)], srcb.at[slot])
                pltpu.sync_copy(dstp.at[pl.ds(base, GRP)], dstb.at[slot])
                for k in range(GRP):
                    pltpu.async_copy(
                        table.at[srcb.at[slot].at[k]], rows.at[slot].at[k],
                        gsem)

            def finish(g, slot, fire_next):
                gsem = gsem0 if slot == 0 else gsem1
                if fire_next:
                    @pl.when(g < NG - 1)
                    def _():
                        load_fire(g + 1, slot ^ 1)
                for k in range(GRP):
                    pltpu.make_async_copy(
                        table.at[srcb.at[slot].at[k]], rows.at[slot].at[k],
                        gsem).wait()
                sds = []
                for k in range(GRP):
                    sds.append(pltpu.async_copy(
                        rows.at[slot].at[k], acc.at[dstb.at[slot].at[k]],
                        ssem, add=True))
                for d in sds:
                    d.wait()

            load_fire(0, 0)

            def body(i, _):
                finish(2 * i, 0, True)
                finish(2 * i + 1, 1, True)
                return 0

            lax.fori_loop(0, (NG - 1) // 2, body, 0)
            finish(NG - 1, 0, False)

        for q in range(2):
            for k in range(8):
                pltpu.sync_copy(zb, acc.at[pl.ds(s * ROWS_PT + k * ZR, ZR)])
            plsc.subcore_barrier()

            @pl.when(c == 0)
            def _():
                run_edges(q)

            @pl.when(c == 1)
            def _():
                run_edges(2 + q)

            plsc.subcore_barrier()

            pltpu.sync_copy(
                acc.at[pl.ds(s * ROWS_PT, ROWS_PT)],
                u_out.at[c, pl.ds(s * ROWS_PT, ROWS_PT), pl.ds(16 * q, 16)])
            plsc.subcore_barrier()

    return agg


def _tc_scale_body(h0_ref, d0_ref, d1_ref, fs_ref, dinv_ref):
    dv = lax.rsqrt(d0_ref[...] + d1_ref[...] + 1.0)
    fs_ref[...] = h0_ref[...] * dv[:, None]
    dinv_ref[...] = dv


def _tc_scale(h0, deg0, deg1):
    return pl.pallas_call(
        _tc_scale_body,
        grid=(NBLK,),
        in_specs=[
            pl.BlockSpec((BLK, EMBED), lambda i: (i, 0)),
            pl.BlockSpec((BLK,), lambda i: (i,)),
            pl.BlockSpec((BLK,), lambda i: (i,)),
        ],
        out_specs=[
            pl.BlockSpec((BLK, EMBED), lambda i: (i, 0)),
            pl.BlockSpec((BLK,), lambda i: (i,)),
        ],
        out_shape=[
            jax.ShapeDtypeStruct((NP, EMBED), jnp.float32),
            jax.ShapeDtypeStruct((NP,), jnp.float32),
        ],
    )(h0, deg0, deg1)


def _tc_mid_body(ulo_ref, uhi_ref, fs_ref, dinv_ref,
                 w1_ref, b1_ref, w2_ref, ts_ref):
    u = jnp.concatenate([ulo_ref[...], uhi_ref[...]], axis=1)
    fs = fs_ref[...]
    dv = dinv_ref[...]
    g1 = (u + fs) * dv[:, None]
    h1 = jnp.dot(g1, w1_ref[...], preferred_element_type=jnp.float32)
    h1 = jnp.maximum(h1 + b1_ref[...][None, :], 0.0)
    t = jnp.dot(h1, w2_ref[...], preferred_element_type=jnp.float32)
    ts_ref[...] = t * dv[:, None]


def _tc_mid(ulo, uhi, fs, dinv, W1, b1, W2):
    return pl.pallas_call(
        _tc_mid_body,
        grid=(NBLK,),
        in_specs=[
            pl.BlockSpec((BLK, 32), lambda i: (i, 0)),
            pl.BlockSpec((BLK, 32), lambda i: (i, 0)),
            pl.BlockSpec((BLK, EMBED), lambda i: (i, 0)),
            pl.BlockSpec((BLK,), lambda i: (i,)),
            pl.BlockSpec((EMBED, H1), lambda i: (0, 0)),
            pl.BlockSpec((H1,), lambda i: (0,)),
            pl.BlockSpec((H1, H2), lambda i: (0, 0)),
        ],
        out_specs=pl.BlockSpec((BLK, H2), lambda i: (i, 0)),
        out_shape=jax.ShapeDtypeStruct((NP, H2), jnp.float32),
    )(ulo, uhi, fs, dinv, W1, b1, W2)


def _tc_pool_body(ulo_ref, uhi_ref, ts_ref, dinv_ref, b2_ref,
                  batch_ref, c0_ref, c1_ref, wfc_ref, bfc_ref,
                  out_ref, segacc):
    i = pl.program_id(0)

    @pl.when(i == 0)
    def _():
        segacc[...] = jnp.zeros_like(segacc)

    u = jnp.concatenate([ulo_ref[...], uhi_ref[...]], axis=1)
    ts = ts_ref[...]
    g2 = (u + ts) * dinv_ref[...][:, None]
    h2 = jnp.maximum(g2 + b2_ref[...][None, :], 0.0)
    b = batch_ref[0, 0, :]
    seg_ids = lax.broadcasted_iota(jnp.int32, (G, BLK), 0)
    mask = (b[None, :] == seg_ids).astype(jnp.float32)
    segacc[...] += jnp.dot(mask, h2, preferred_element_type=jnp.float32)

    @pl.when(i == pl.num_programs(0) - 1)
    def _():
        cnt = (c0_ref[...] + c1_ref[...])[:G]
        pooled = segacc[...] / jnp.maximum(cnt, 1.0)[:, None]
        res = jnp.dot(pooled, wfc_ref[...], preferred_element_type=jnp.float32)
        out_ref[...] = jax.nn.sigmoid(res + bfc_ref[0, 0])


def _tc_pool(ulo, uhi, ts, dinv, b2, batch3, c0, c1, Wfcp, bfc2):
    return pl.pallas_call(
        _tc_pool_body,
        grid=(NBLK,),
        in_specs=[
            pl.BlockSpec((BLK, 32), lambda i: (i, 0)),
            pl.BlockSpec((BLK, 32), lambda i: (i, 0)),
            pl.BlockSpec((BLK, H2), lambda i: (i, 0)),
            pl.BlockSpec((BLK,), lambda i: (i,)),
            pl.BlockSpec((H2,), lambda i: (0,)),
            pl.BlockSpec((1, 1, BLK), lambda i: (i, 0, 0)),
            pl.BlockSpec((GP,), lambda i: (0,)),
            pl.BlockSpec((GP,), lambda i: (0,)),
            pl.BlockSpec((H2, LANE), lambda i: (0, 0)),
            pl.BlockSpec(memory_space=pltpu.SMEM),
        ],
        out_specs=pl.BlockSpec((G, LANE), lambda i: (0, 0)),
        out_shape=jax.ShapeDtypeStruct((G, LANE), jnp.float32),
        scratch_shapes=[pltpu.VMEM((G, H2), jnp.float32)],
    )(ulo, uhi, ts, dinv, b2, batch3, c0, c1, Wfcp, bfc2)


_prep = _make_prep()
_agg = _make_agg()


@jax.jit
def kernel(x, edge_index, batch, emb, W1, b1, W2, b2, Wfc, bfc):
    xi = x[:, 0].astype(jnp.int32)
    xi_p = jnp.concatenate(
        [xi, jnp.zeros((NP - N,), jnp.int32)]).reshape(NODE_CH * 32, LANE)
    batch_p = jnp.concatenate(
        [batch.astype(jnp.int32), jnp.full((NP - N,), GDUMP, jnp.int32)])
    batch2d = batch_p.reshape(NODE_CH * 32, LANE)
    batch3 = batch_p.reshape(NBLK, 1, BLK)
    src4_p = jnp.concatenate(
        [edge_index[0].astype(jnp.int32) * 4, jnp.zeros((EP - E,), jnp.int32)]
    ).reshape(ECH, LANE)
    dst_p = jnp.concatenate(
        [edge_index[1].astype(jnp.int32), jnp.full((EP - E,), NDUMP, jnp.int32)]
    ).reshape(ECH, LANE)

    h0, deg, cnt = _prep(emb, xi_p, dst_p, batch2d)
    fs, dinv = _tc_scale(h0, deg[0], deg[1])
    u1 = _agg(fs.reshape(4 * NP, 16), src4_p, dst_p)
    ts = _tc_mid(u1[0], u1[1], fs, dinv, W1, b1, W2)
    u2 = _agg(ts.reshape(4 * NP, 16), src4_p, dst_p)
    Wfcp = jnp.pad(Wfc, ((0, 0), (0, LANE - 1)))
    bfc2 = bfc.reshape(1, 1)
    outp = _tc_pool(u2[0], u2[1], ts, dinv, b2, batch3,
                    cnt[0], cnt[1], Wfcp, bfc2)
    return outp[:, 0]


# speedup vs baseline: 24.1307x; 1.0529x over previous
import functools

import jax
import jax.numpy as jnp
from jax import lax
from jax.experimental import pallas as pl
from jax.experimental.pallas import tpu as pltpu
from jax.experimental.pallas import tpu_sc as plsc

N = 50000
E = 800000
VOCAB = 100000
EMBED = 64
H1 = 128
H2 = 64
G = 1024

NC = 2
NS = 16
LANE = 128

NP = 53248
NODE_CH = 13
EP = 802816
ECH = 6272
ECH_PT32 = ECH // 32
ECH_PT16 = ECH // 16
NDUMP = N
GP = 1152
GDUMP = G
ROWS_PT = NP // NS
ZR = 416
GRP = 14

BLK = 2048
NBLK = NP // BLK

_mesh = plsc.VectorSubcoreMesh(core_axis_name="c", subcore_axis_name="s")


def _zero_fill_1d(buf, nwords):
    z = jnp.zeros((16,), jnp.float32)

    def body(i, _):
        buf[pl.ds(i * 16, 16)] = z
        return 0

    lax.fori_loop(0, nwords // 16, body, 0)


def _zero_fill_2d(buf, nrows):
    z = jnp.zeros((16,), jnp.float32)

    def body(i, _):
        buf[i, pl.ds(0, 16)] = z
        return 0

    lax.fori_loop(0, nrows, body, 0)


def _make_prep():
    @functools.partial(
        pl.kernel,
        out_type=(
            jax.ShapeDtypeStruct((NP, EMBED), jnp.float32),
            jax.ShapeDtypeStruct((NC, NP), jnp.float32),
            jax.ShapeDtypeStruct((NC, GP), jnp.float32),
        ),
        mesh=_mesh,
        compiler_params=pltpu.CompilerParams(use_tc_tiling_on_sc=False),
        scratch_types=(
            pltpu.VMEM((8, LANE), jnp.int32),
            pltpu.VMEM((NODE_CH, LANE), jnp.int32),
            pltpu.VMEM((NODE_CH * LANE, EMBED), jnp.float32),
            pltpu.VMEM((LANE,), jnp.float32),
            pltpu.VMEM((ROWS_PT,), jnp.float32),
            pltpu.VMEM_SHARED((NP,), jnp.float32),
            pltpu.VMEM_SHARED((GP,), jnp.float32),
            pltpu.SemaphoreType.DMA,
            pltpu.SemaphoreType.DMA,
            pltpu.SemaphoreType.DMA,
        ),
    )
    def prep(emb, xi, dstp, batchp, h0_out, deg_out, cnt_out,
             idxb, idxb2, rows, onesb, zb, dega, cnta,
             gsem, ssem, wsem):
        c = lax.axis_index("c")
        s = lax.axis_index("s")
        wid = c * NS + s

        _zero_fill_1d(zb, ROWS_PT)
        o = jnp.ones((16,), jnp.float32)
        for i in range(8):
            onesb[pl.ds(i * 16, 16)] = o

        pltpu.sync_copy(zb, dega.at[pl.ds(s * ROWS_PT, ROWS_PT)])

        @pl.when(s == 0)
        def _():
            pltpu.sync_copy(zb.at[pl.ds(0, GP)], cnta)

        plsc.subcore_barrier()

        pltpu.sync_copy(xi.at[pl.ds(wid * NODE_CH, NODE_CH)], idxb2)
        gds = []
        for k in range(NODE_CH):
            gds.append(pltpu.async_copy(
                emb.at[idxb2.at[k]], rows.at[pl.ds(k * LANE, LANE)], gsem))

        def deg_grp(base, n):
            pltpu.sync_copy(dstp.at[pl.ds(base, n)], idxb.at[pl.ds(0, n)])
            ds_ = []
            for k in range(n):
                ds_.append(pltpu.async_copy(
                    onesb, dega.at[idxb.at[k]], ssem, add=True))
            for d in ds_:
                d.wait()

        def deg_body(g, _):
            deg_grp(wid * ECH_PT32 + g * 8, 8)
            return 0

        lax.fori_loop(0, ECH_PT32 // 8, deg_body, 0)
        deg_grp(wid * ECH_PT32 + (ECH_PT32 // 8) * 8, ECH_PT32 % 8)

        pltpu.sync_copy(batchp.at[pl.ds(wid * NODE_CH, 8)], idxb)
        cds = []
        for k in range(8):
            cds.append(pltpu.async_copy(
                onesb, cnta.at[idxb.at[k]], ssem, add=True))
        for d in cds:
            d.wait()
        pltpu.sync_copy(batchp.at[pl.ds(wid * NODE_CH + 8, NODE_CH - 8)],
                        idxb.at[pl.ds(0, NODE_CH - 8)])
        cds = []
        for k in range(NODE_CH - 8):
            cds.append(pltpu.async_copy(
                onesb, cnta.at[idxb.at[k]], ssem, add=True))
        for d in cds:
            d.wait()

        for d in gds:
            d.wait()

        plsc.subcore_barrier()

        pltpu.sync_copy(dega.at[pl.ds(s * ROWS_PT, ROWS_PT)],
                        deg_out.at[c, pl.ds(s * ROWS_PT, ROWS_PT)])

        wds = []
        for k in range(NODE_CH):
            wds.append(pltpu.async_copy(
                rows.at[pl.ds(k * LANE, LANE)],
                h0_out.at[pl.ds((wid * NODE_CH + k) * LANE, LANE)], wsem))
        for d in wds:
            d.wait()

        @pl.when(s == 0)
        def _():
            pltpu.sync_copy(cnta, cnt_out.at[c])

    return prep


def _make_agg():
    @functools.partial(
        pl.kernel,
        out_type=jax.ShapeDtypeStruct((NC, NP, 32), jnp.float32),
        mesh=_mesh,
        compiler_params=pltpu.CompilerParams(use_tc_tiling_on_sc=False),
        scratch_types=(
            pltpu.VMEM((2, GRP, LANE), jnp.int32),
            pltpu.VMEM((2, GRP, LANE), jnp.int32),
            pltpu.VMEM((2, GRP, LANE, 16), jnp.float32),
            pltpu.VMEM((ZR, 16), jnp.float32),
            pltpu.VMEM_SHARED((NP, 16), jnp.float32),
            pltpu.SemaphoreType.DMA,
            pltpu.SemaphoreType.DMA,
            pltpu.SemaphoreType.DMA,
        ),
    )
    def agg(tbl4, srcp4, dstp, u_out,
            srcb, dstb, rows, zb, acc, gsem0, gsem1, ssem):
        c = lax.axis_index("c")
        s = lax.axis_index("s")
        NG = ECH_PT16 // GRP

        _zero_fill_2d(zb, ZR)

        def run_edges(qoff):
            table = tbl4.at[pl.ds(qoff, 4 * NP - 3)]

            def load_fire(g, slot):
                gsem = gsem0 if slot == 0 else gsem1
                base = s * ECH_PT16 + g * GRP
                pltpu.sync_copy(srcp4.at[pl.ds(base, GRP)], srcb.at[slot])
                pltpu.sync_copy(dstp.at[pl.ds(base, GRP)], dstb.at[slot])
                for k in range(GRP):
                    pltpu.async_copy(
                        table.at[srcb.at[slot].at[k]], rows.at[slot].at[k],
                        gsem)

            def finish(g, slot, fire_next):
                gsem = gsem0 if slot == 0 else gsem1
                if fire_next:
                    load_fire(g + 1, slot ^ 1)
                for k in range(GRP):
                    pltpu.make_async_copy(
                        table.at[srcb.at[slot].at[k]], rows.at[slot].at[k],
                        gsem).wait()
                sds = []
                for k in range(GRP):
                    sds.append(pltpu.async_copy(
                        rows.at[slot].at[k], acc.at[dstb.at[slot].at[k]],
                        ssem, add=True))
                for d in sds:
                    d.wait()

            load_fire(0, 0)
            nbody = (NG - 1) // 2

            def body(i, _):
                finish(2 * i, 0, True)
                finish(2 * i + 1, 1, True)
                return 0

            lax.fori_loop(0, nbody, body, 0)
            for g in range(2 * nbody, NG):
                finish(g, g % 2, g < NG - 1)

        for q in range(2):
            for k in range(8):
                pltpu.sync_copy(zb, acc.at[pl.ds(s * ROWS_PT + k * ZR, ZR)])
            plsc.subcore_barrier()

            @pl.when(c == 0)
            def _():
                run_edges(q)

            @pl.when(c == 1)
            def _():
                run_edges(2 + q)

            plsc.subcore_barrier()

            pltpu.sync_copy(
                acc.at[pl.ds(s * ROWS_PT, ROWS_PT)],
                u_out.at[c, pl.ds(s * ROWS_PT, ROWS_PT), pl.ds(16 * q, 16)])
            plsc.subcore_barrier()

    return agg


def _tc_scale_body(h0_ref, d0_ref, d1_ref, fs_ref, dinv_ref):
    dv = lax.rsqrt(d0_ref[...] + d1_ref[...] + 1.0)
    fs_ref[...] = h0_ref[...] * dv[:, None]
    dinv_ref[...] = dv


def _tc_scale(h0, deg0, deg1):
    return pl.pallas_call(
        _tc_scale_body,
        grid=(NBLK,),
        in_specs=[
            pl.BlockSpec((BLK, EMBED), lambda i: (i, 0)),
            pl.BlockSpec((BLK,), lambda i: (i,)),
            pl.BlockSpec((BLK,), lambda i: (i,)),
        ],
        out_specs=[
            pl.BlockSpec((BLK, EMBED), lambda i: (i, 0)),
            pl.BlockSpec((BLK,), lambda i: (i,)),
        ],
        out_shape=[
            jax.ShapeDtypeStruct((NP, EMBED), jnp.float32),
            jax.ShapeDtypeStruct((NP,), jnp.float32),
        ],
    )(h0, deg0, deg1)


def _tc_mid_body(ulo_ref, uhi_ref, fs_ref, dinv_ref,
                 w1_ref, b1_ref, w2_ref, ts_ref):
    u = jnp.concatenate([ulo_ref[...], uhi_ref[...]], axis=1)
    fs = fs_ref[...]
    dv = dinv_ref[...]
    g1 = (u + fs) * dv[:, None]
    h1 = jnp.dot(g1, w1_ref[...], preferred_element_type=jnp.float32)
    h1 = jnp.maximum(h1 + b1_ref[...][None, :], 0.0)
    t = jnp.dot(h1, w2_ref[...], preferred_element_type=jnp.float32)
    ts_ref[...] = t * dv[:, None]


def _tc_mid(ulo, uhi, fs, dinv, W1, b1, W2):
    return pl.pallas_call(
        _tc_mid_body,
        grid=(NBLK,),
        in_specs=[
            pl.BlockSpec((BLK, 32), lambda i: (i, 0)),
            pl.BlockSpec((BLK, 32), lambda i: (i, 0)),
            pl.BlockSpec((BLK, EMBED), lambda i: (i, 0)),
            pl.BlockSpec((BLK,), lambda i: (i,)),
            pl.BlockSpec((EMBED, H1), lambda i: (0, 0)),
            pl.BlockSpec((H1,), lambda i: (0,)),
            pl.BlockSpec((H1, H2), lambda i: (0, 0)),
        ],
        out_specs=pl.BlockSpec((BLK, H2), lambda i: (i, 0)),
        out_shape=jax.ShapeDtypeStruct((NP, H2), jnp.float32),
    )(ulo, uhi, fs, dinv, W1, b1, W2)


def _tc_pool_body(ulo_ref, uhi_ref, ts_ref, dinv_ref, b2_ref,
                  batch_ref, c0_ref, c1_ref, wfc_ref, bfc_ref,
                  out_ref, segacc):
    i = pl.program_id(0)

    @pl.when(i == 0)
    def _():
        segacc[...] = jnp.zeros_like(segacc)

    u = jnp.concatenate([ulo_ref[...], uhi_ref[...]], axis=1)
    ts = ts_ref[...]
    g2 = (u + ts) * dinv_ref[...][:, None]
    h2 = jnp.maximum(g2 + b2_ref[...][None, :], 0.0)
    b = batch_ref[0, 0, :]
    seg_ids = lax.broadcasted_iota(jnp.int32, (G, BLK), 0)
    mask = (b[None, :] == seg_ids).astype(jnp.float32)
    segacc[...] += jnp.dot(mask, h2, preferred_element_type=jnp.float32)

    @pl.when(i == pl.num_programs(0) - 1)
    def _():
        cnt = (c0_ref[...] + c1_ref[...])[:G]
        pooled = segacc[...] / jnp.maximum(cnt, 1.0)[:, None]
        res = jnp.dot(pooled, wfc_ref[...], preferred_element_type=jnp.float32)
        out_ref[...] = jax.nn.sigmoid(res + bfc_ref[0, 0])


def _tc_pool(ulo, uhi, ts, dinv, b2, batch3, c0, c1, Wfcp, bfc2):
    return pl.pallas_call(
        _tc_pool_body,
        grid=(NBLK,),
        in_specs=[
            pl.BlockSpec((BLK, 32), lambda i: (i, 0)),
            pl.BlockSpec((BLK, 32), lambda i: (i, 0)),
            pl.BlockSpec((BLK, H2), lambda i: (i, 0)),
            pl.BlockSpec((BLK,), lambda i: (i,)),
            pl.BlockSpec((H2,), lambda i: (0,)),
            pl.BlockSpec((1, 1, BLK), lambda i: (i, 0, 0)),
            pl.BlockSpec((GP,), lambda i: (0,)),
            pl.BlockSpec((GP,), lambda i: (0,)),
            pl.BlockSpec((H2, LANE), lambda i: (0, 0)),
            pl.BlockSpec(memory_space=pltpu.SMEM),
        ],
        out_specs=pl.BlockSpec((G, LANE), lambda i: (0, 0)),
        out_shape=jax.ShapeDtypeStruct((G, LANE), jnp.float32),
        scratch_shapes=[pltpu.VMEM((G, H2), jnp.float32)],
    )(ulo, uhi, ts, dinv, b2, batch3, c0, c1, Wfcp, bfc2)


_prep = _make_prep()
_agg = _make_agg()


@jax.jit
def kernel(x, edge_index, batch, emb, W1, b1, W2, b2, Wfc, bfc):
    xi = x[:, 0].astype(jnp.int32)
    xi_p = jnp.concatenate(
        [xi, jnp.zeros((NP - N,), jnp.int32)]).reshape(NODE_CH * 32, LANE)
    batch_p = jnp.concatenate(
        [batch.astype(jnp.int32), jnp.full((NP - N,), GDUMP, jnp.int32)])
    batch2d = batch_p.reshape(NODE_CH * 32, LANE)
    batch3 = batch_p.reshape(NBLK, 1, BLK)
    src4_p = jnp.concatenate(
        [edge_index[0].astype(jnp.int32) * 4, jnp.zeros((EP - E,), jnp.int32)]
    ).reshape(ECH, LANE)
    dst_p = jnp.concatenate(
        [edge_index[1].astype(jnp.int32), jnp.full((EP - E,), NDUMP, jnp.int32)]
    ).reshape(ECH, LANE)

    h0, deg, cnt = _prep(emb, xi_p, dst_p, batch2d)
    fs, dinv = _tc_scale(h0, deg[0], deg[1])
    u1 = _agg(fs.reshape(4 * NP, 16), src4_p, dst_p)
    ts = _tc_mid(u1[0], u1[1], fs, dinv, W1, b1, W2)
    u2 = _agg(ts.reshape(4 * NP, 16), src4_p, dst_p)
    Wfcp = jnp.pad(Wfc, ((0, 0), (0, LANE - 1)))
    bfc2 = bfc.reshape(1, 1)
    outp = _tc_pool(u2[0], u2[1], ts, dinv, b2, batch3,
                    cnt[0], cnt[1], Wfcp, bfc2)
    return outp[:, 0]
